# baseline (device time: 179120 ns/iter reference)
import jax
import jax.numpy as jnp
from jax import lax
from jax.experimental import pallas as pl
from jax.experimental.pallas import tpu as pltpu

N_DEV = 4


def _gelu(z):
    return 0.5 * z * (1.0 + jnp.tanh(0.7978845608 * (z + 0.044715 * z * z * z)))


def kernel(A, B):
    m, k_per = A.shape
    _, n = B.shape
    m_chunk = m // N_DEV

    def body(a_ref, b_ref, out_ref, partial_ref, rs_ref, ag_ref,
             rs_send, rs_recv, ag_send, ag_recv):
        my = lax.axis_index("i")
        left = lax.rem(my + N_DEV - 1, N_DEV)
        right = lax.rem(my + 1, N_DEV)

        def chunk_rows(c):
            return pl.ds(c * m_chunk, m_chunk)

        barrier_sem = pltpu.get_barrier_semaphore()
        for nbr in (left, right):
            pl.semaphore_signal(
                barrier_sem, inc=1,
                device_id=(nbr,), device_id_type=pl.DeviceIdType.MESH,
            )
        pl.semaphore_wait(barrier_sem, 2)

        partial_ref[:, :] = jnp.dot(
            a_ref[:, :], b_ref[:, :], preferred_element_type=jnp.float32
        )

        for s in range(N_DEV - 1):
            c_send = lax.rem(my - s + N_DEV, N_DEV)
            c_recv = lax.rem(my - s - 1 + N_DEV, N_DEV)
            src = (partial_ref.at[chunk_rows(c_send), :] if s == 0
                   else rs_ref.at[s - 1])
            rdma = pltpu.make_async_remote_copy(
                src_ref=src,
                dst_ref=rs_ref.at[s],
                send_sem=rs_send.at[s],
                recv_sem=rs_recv.at[s],
                device_id=(right,),
                device_id_type=pl.DeviceIdType.MESH,
            )
            rdma.start()
            rdma.wait()
            rs_ref[s, :, :] = rs_ref[s, :, :] + partial_ref[chunk_rows(c_recv), :]

        c_own = lax.rem(my + 1, N_DEV)
        rs_ref[N_DEV - 2, :, :] = _gelu(rs_ref[N_DEV - 2, :, :])
        out_ref[chunk_rows(c_own), :] = rs_ref[N_DEV - 2, :, :]

        for h in range(N_DEV - 1):
            src = (rs_ref.at[N_DEV - 2] if h == 0 else ag_ref.at[h - 1])
            rdma = pltpu.make_async_remote_copy(
                src_ref=src,
                dst_ref=ag_ref.at[h],
                send_sem=ag_send.at[h],
                recv_sem=ag_recv.at[h],
                device_id=(right,),
                device_id_type=pl.DeviceIdType.MESH,
            )
            rdma.start()
            rdma.wait()
            c_got = lax.rem(my - h + N_DEV, N_DEV)
            out_ref[chunk_rows(c_got), :] = ag_ref[h, :, :]

    return pl.pallas_call(
        body,
        out_shape=jax.ShapeDtypeStruct((m, n), jnp.float32),
        in_specs=[
            pl.BlockSpec(memory_space=pltpu.VMEM),
            pl.BlockSpec(memory_space=pltpu.VMEM),
        ],
        out_specs=pl.BlockSpec(memory_space=pltpu.VMEM),
        scratch_shapes=[
            pltpu.VMEM((m, n), jnp.float32),
            pltpu.VMEM((N_DEV - 1, m_chunk, n), jnp.float32),
            pltpu.VMEM((N_DEV - 1, m_chunk, n), jnp.float32),
            pltpu.SemaphoreType.DMA((N_DEV - 1,)),
            pltpu.SemaphoreType.DMA((N_DEV - 1,)),
            pltpu.SemaphoreType.DMA((N_DEV - 1,)),
            pltpu.SemaphoreType.DMA((N_DEV - 1,)),
        ],
        compiler_params=pltpu.CompilerParams(collective_id=0),
    )(A, B)


# device time: 100415 ns/iter; 1.7838x vs baseline; 1.7838x over previous
import jax
import jax.numpy as jnp
from jax import lax
from jax.experimental import pallas as pl
from jax.experimental.pallas import tpu as pltpu

N_DEV = 4


def _gelu(z):
    return 0.5 * z * (1.0 + jnp.tanh(0.7978845608 * (z + 0.044715 * z * z * z)))


def kernel(A, B):
    m, k_per = A.shape
    _, n = B.shape
    m_chunk = m // N_DEV
    n_half = n // 2

    def body(a_ref, b_ref, out_ref, partial_ref, rs_r, rs_l, ag_r, ag_l,
             rs_r_send, rs_r_recv, rs_l_send, rs_l_recv,
             ag_r_send, ag_r_recv, ag_l_send, ag_l_recv):
        my = lax.axis_index("i")
        left = lax.rem(my + N_DEV - 1, N_DEV)
        right = lax.rem(my + 1, N_DEV)

        def mod4(x):
            return lax.rem(x + 2 * N_DEV, N_DEV)

        def rows(c):
            return pl.ds(c * m_chunk, m_chunk)

        lo = pl.ds(0, n_half)
        hi = pl.ds(n_half, n_half)

        def mm(c, col_lo):
            a_chunk = a_ref[rows(c), :]
            b_half = b_ref[:, 0:n_half] if col_lo else b_ref[:, n_half:n]
            partial_ref[rows(c), lo if col_lo else hi] = jnp.dot(
                a_chunk, b_half, preferred_element_type=jnp.float32
            )

        barrier_sem = pltpu.get_barrier_semaphore()
        for nbr in (left, right):
            pl.semaphore_signal(
                barrier_sem, inc=1,
                device_id=(nbr,), device_id_type=pl.DeviceIdType.MESH,
            )
        pl.semaphore_wait(barrier_sem, 2)

        mm(my, True)
        mm(my, False)

        for s in range(N_DEV - 1):
            r = pltpu.make_async_remote_copy(
                src_ref=(partial_ref.at[rows(my), lo] if s == 0
                         else rs_r.at[s - 1]),
                dst_ref=rs_r.at[s],
                send_sem=rs_r_send.at[s], recv_sem=rs_r_recv.at[s],
                device_id=(right,), device_id_type=pl.DeviceIdType.MESH,
            )
            l = pltpu.make_async_remote_copy(
                src_ref=(partial_ref.at[rows(my), hi] if s == 0
                         else rs_l.at[s - 1]),
                dst_ref=rs_l.at[s],
                send_sem=rs_l_send.at[s], recv_sem=rs_l_recv.at[s],
                device_id=(left,), device_id_type=pl.DeviceIdType.MESH,
            )
            r.start()
            l.start()
            c_r = mod4(my - s - 1)
            c_l = mod4(my + s + 1)
            mm(c_r, True)
            mm(c_l, False)
            r.wait()
            l.wait()
            rs_r[s, :, :] = rs_r[s, :, :] + partial_ref[rows(c_r), lo]
            rs_l[s, :, :] = rs_l[s, :, :] + partial_ref[rows(c_l), hi]

        rs_r[N_DEV - 2, :, :] = _gelu(rs_r[N_DEV - 2, :, :])
        rs_l[N_DEV - 2, :, :] = _gelu(rs_l[N_DEV - 2, :, :])
        out_ref[rows(mod4(my + 1)), lo] = rs_r[N_DEV - 2, :, :]
        out_ref[rows(mod4(my - 1)), hi] = rs_l[N_DEV - 2, :, :]

        for h in range(N_DEV - 1):
            r = pltpu.make_async_remote_copy(
                src_ref=(rs_r.at[N_DEV - 2] if h == 0 else ag_r.at[h - 1]),
                dst_ref=ag_r.at[h],
                send_sem=ag_r_send.at[h], recv_sem=ag_r_recv.at[h],
                device_id=(right,), device_id_type=pl.DeviceIdType.MESH,
            )
            l = pltpu.make_async_remote_copy(
                src_ref=(rs_l.at[N_DEV - 2] if h == 0 else ag_l.at[h - 1]),
                dst_ref=ag_l.at[h],
                send_sem=ag_l_send.at[h], recv_sem=ag_l_recv.at[h],
                device_id=(left,), device_id_type=pl.DeviceIdType.MESH,
            )
            r.start()
            l.start()
            r.wait()
            l.wait()
            out_ref[rows(mod4(my - h)), lo] = ag_r[h, :, :]
            out_ref[rows(mod4(my + h)), hi] = ag_l[h, :, :]

    comm_shape = (N_DEV - 1, m_chunk, n_half)
    return pl.pallas_call(
        body,
        out_shape=jax.ShapeDtypeStruct((m, n), jnp.float32),
        in_specs=[
            pl.BlockSpec(memory_space=pltpu.VMEM),
            pl.BlockSpec(memory_space=pltpu.VMEM),
        ],
        out_specs=pl.BlockSpec(memory_space=pltpu.VMEM),
        scratch_shapes=[
            pltpu.VMEM((m, n), jnp.float32),
            pltpu.VMEM(comm_shape, jnp.float32),
            pltpu.VMEM(comm_shape, jnp.float32),
            pltpu.VMEM(comm_shape, jnp.float32),
            pltpu.VMEM(comm_shape, jnp.float32),
            pltpu.SemaphoreType.DMA((N_DEV - 1,)),
            pltpu.SemaphoreType.DMA((N_DEV - 1,)),
            pltpu.SemaphoreType.DMA((N_DEV - 1,)),
            pltpu.SemaphoreType.DMA((N_DEV - 1,)),
            pltpu.SemaphoreType.DMA((N_DEV - 1,)),
            pltpu.SemaphoreType.DMA((N_DEV - 1,)),
            pltpu.SemaphoreType.DMA((N_DEV - 1,)),
            pltpu.SemaphoreType.DMA((N_DEV - 1,)),
        ],
        compiler_params=pltpu.CompilerParams(collective_id=0),
    )(A, B)


# device time: 62445 ns/iter; 2.8684x vs baseline; 1.6081x over previous
import jax
import jax.numpy as jnp
from jax import lax
from jax.experimental import pallas as pl
from jax.experimental.pallas import tpu as pltpu

N_DEV = 4


def _gelu(z):
    return 0.5 * z * (1.0 + jnp.tanh(0.7978845608 * (z + 0.044715 * z * z * z)))


def kernel(A, B):
    m, k_per = A.shape
    _, n = B.shape
    m_chunk = m // N_DEV
    n_half = n // 2

    def body(a_ref, b_ref, out_ref, partial_ref,
             srs_r, srs_l, rs_r, rs_l, gsb_r, gsb_l, ag_r, ag_l,
             rs_r_send, rs_r_recv, rs_l_send, rs_l_recv,
             ag_r_send, ag_r_recv, ag_l_send, ag_l_recv):
        my = lax.axis_index("i")
        left = lax.rem(my + N_DEV - 1, N_DEV)
        right = lax.rem(my + 1, N_DEV)

        def mod4(x):
            return lax.rem(x + 2 * N_DEV, N_DEV)

        def rows(c):
            return pl.ds(c * m_chunk, m_chunk)

        lo = pl.ds(0, n_half)
        hi = pl.ds(n_half, n_half)

        def mm(c, col_lo):
            a_chunk = a_ref[rows(c), :]
            b_half = b_ref[:, 0:n_half] if col_lo else b_ref[:, n_half:n]
            partial_ref[rows(c), lo if col_lo else hi] = jnp.dot(
                a_chunk, b_half, preferred_element_type=jnp.float32
            )

        barrier_sem = pltpu.get_barrier_semaphore()
        for nbr in (left, right):
            pl.semaphore_signal(
                barrier_sem, inc=1,
                device_id=(nbr,), device_id_type=pl.DeviceIdType.MESH,
            )
        pl.semaphore_wait(barrier_sem, 2)

        mm(my, True)
        mm(my, False)
        srs_r[0, :, :] = partial_ref[rows(my), lo].astype(jnp.bfloat16)
        srs_l[0, :, :] = partial_ref[rows(my), hi].astype(jnp.bfloat16)

        for s in range(N_DEV - 1):
            r = pltpu.make_async_remote_copy(
                src_ref=srs_r.at[s], dst_ref=rs_r.at[s],
                send_sem=rs_r_send.at[s], recv_sem=rs_r_recv.at[s],
                device_id=(right,), device_id_type=pl.DeviceIdType.MESH,
            )
            l = pltpu.make_async_remote_copy(
                src_ref=srs_l.at[s], dst_ref=rs_l.at[s],
                send_sem=rs_l_send.at[s], recv_sem=rs_l_recv.at[s],
                device_id=(left,), device_id_type=pl.DeviceIdType.MESH,
            )
            r.start()
            l.start()
            c_r = mod4(my - s - 1)
            c_l = mod4(my + s + 1)
            mm(c_r, True)
            mm(c_l, False)
            r.wait()
            l.wait()
            acc_r = rs_r[s, :, :].astype(jnp.float32) + partial_ref[rows(c_r), lo]
            acc_l = rs_l[s, :, :].astype(jnp.float32) + partial_ref[rows(c_l), hi]
            if s < N_DEV - 2:
                srs_r[s + 1, :, :] = acc_r.astype(jnp.bfloat16)
                srs_l[s + 1, :, :] = acc_l.astype(jnp.bfloat16)
            else:
                g_r = _gelu(acc_r)
                g_l = _gelu(acc_l)
                out_ref[rows(mod4(my + 1)), lo] = g_r
                out_ref[rows(mod4(my - 1)), hi] = g_l
                gsb_r[:, :] = g_r.astype(jnp.bfloat16)
                gsb_l[:, :] = g_l.astype(jnp.bfloat16)

        rd = []
        ld = []
        for h in range(N_DEV - 1):
            rd.append(pltpu.make_async_remote_copy(
                src_ref=(gsb_r if h == 0 else ag_r.at[h - 1]),
                dst_ref=ag_r.at[h],
                send_sem=ag_r_send.at[h], recv_sem=ag_r_recv.at[h],
                device_id=(right,), device_id_type=pl.DeviceIdType.MESH,
            ))
            ld.append(pltpu.make_async_remote_copy(
                src_ref=(gsb_l if h == 0 else ag_l.at[h - 1]),
                dst_ref=ag_l.at[h],
                send_sem=ag_l_send.at[h], recv_sem=ag_l_recv.at[h],
                device_id=(left,), device_id_type=pl.DeviceIdType.MESH,
            ))
        rd[0].start()
        ld[0].start()
        for h in range(N_DEV - 1):
            rd[h].wait()
            ld[h].wait()
            if h < N_DEV - 2:
                rd[h + 1].start()
                ld[h + 1].start()
            out_ref[rows(mod4(my - h)), lo] = ag_r[h, :, :].astype(jnp.float32)
            out_ref[rows(mod4(my + h)), hi] = ag_l[h, :, :].astype(jnp.float32)

    comm_shape = (N_DEV - 1, m_chunk, n_half)
    return pl.pallas_call(
        body,
        out_shape=jax.ShapeDtypeStruct((m, n), jnp.float32),
        in_specs=[
            pl.BlockSpec(memory_space=pltpu.VMEM),
            pl.BlockSpec(memory_space=pltpu.VMEM),
        ],
        out_specs=pl.BlockSpec(memory_space=pltpu.VMEM),
        scratch_shapes=[
            pltpu.VMEM((m, n), jnp.float32),
            pltpu.VMEM(comm_shape, jnp.bfloat16),
            pltpu.VMEM(comm_shape, jnp.bfloat16),
            pltpu.VMEM(comm_shape, jnp.bfloat16),
            pltpu.VMEM(comm_shape, jnp.bfloat16),
            pltpu.VMEM((m_chunk, n_half), jnp.bfloat16),
            pltpu.VMEM((m_chunk, n_half), jnp.bfloat16),
            pltpu.VMEM(comm_shape, jnp.bfloat16),
            pltpu.VMEM(comm_shape, jnp.bfloat16),
            pltpu.SemaphoreType.DMA((N_DEV - 1,)),
            pltpu.SemaphoreType.DMA((N_DEV - 1,)),
            pltpu.SemaphoreType.DMA((N_DEV - 1,)),
            pltpu.SemaphoreType.DMA((N_DEV - 1,)),
            pltpu.SemaphoreType.DMA((N_DEV - 1,)),
            pltpu.SemaphoreType.DMA((N_DEV - 1,)),
            pltpu.SemaphoreType.DMA((N_DEV - 1,)),
            pltpu.SemaphoreType.DMA((N_DEV - 1,)),
        ],
        compiler_params=pltpu.CompilerParams(collective_id=0),
    )(A, B)


# device time: 61325 ns/iter; 2.9208x vs baseline; 1.0183x over previous
import jax
import jax.numpy as jnp
from jax import lax
from jax.experimental import pallas as pl
from jax.experimental.pallas import tpu as pltpu

N_DEV = 4


def _gelu(z):
    return 0.5 * z * (1.0 + jnp.tanh(0.7978845608 * (z + 0.044715 * z * z * z)))


def kernel(A, B):
    m, k_per = A.shape
    _, n = B.shape
    m_chunk = m // N_DEV
    n_half = n // 2

    def body(a_ref, b_ref, out_ref, partial_ref,
             srs_r, srs_l, rs_r, rs_l, gsb_r, gsb_l, ag_r, ag_l,
             rs_r_send, rs_r_recv, rs_l_send, rs_l_recv,
             ag_r_send, ag_r_recv, ag_l_send, ag_l_recv):
        my = lax.axis_index("i")
        left = lax.rem(my + N_DEV - 1, N_DEV)
        right = lax.rem(my + 1, N_DEV)

        def mod4(x):
            return lax.rem(x + 2 * N_DEV, N_DEV)

        def rows(c):
            return pl.ds(c * m_chunk, m_chunk)

        lo = pl.ds(0, n_half)
        hi = pl.ds(n_half, n_half)

        def mm(c, col_lo):
            a_chunk = a_ref[rows(c), :]
            b_half = b_ref[:, 0:n_half] if col_lo else b_ref[:, n_half:n]
            partial_ref[rows(c), lo if col_lo else hi] = jnp.dot(
                a_chunk, b_half, preferred_element_type=jnp.float32
            )

        barrier_sem = pltpu.get_barrier_semaphore()
        for nbr in (left, right):
            pl.semaphore_signal(
                barrier_sem, inc=1,
                device_id=(nbr,), device_id_type=pl.DeviceIdType.MESH,
            )
        pl.semaphore_wait(barrier_sem, 2)

        mm(my, True)
        mm(my, False)
        srs_r[0, :, :] = partial_ref[rows(my), lo].astype(jnp.bfloat16)
        srs_l[0, :, :] = partial_ref[rows(my), hi].astype(jnp.bfloat16)

        for s in range(N_DEV - 1):
            r = pltpu.make_async_remote_copy(
                src_ref=srs_r.at[s], dst_ref=rs_r.at[s],
                send_sem=rs_r_send.at[s], recv_sem=rs_r_recv.at[s],
                device_id=(right,), device_id_type=pl.DeviceIdType.MESH,
            )
            l = pltpu.make_async_remote_copy(
                src_ref=srs_l.at[s], dst_ref=rs_l.at[s],
                send_sem=rs_l_send.at[s], recv_sem=rs_l_recv.at[s],
                device_id=(left,), device_id_type=pl.DeviceIdType.MESH,
            )
            r.start()
            l.start()
            c_r = mod4(my - s - 1)
            c_l = mod4(my + s + 1)
            mm(c_r, True)
            mm(c_l, False)
            r.wait()
            l.wait()
            acc_r = rs_r[s, :, :].astype(jnp.float32) + partial_ref[rows(c_r), lo]
            acc_l = rs_l[s, :, :].astype(jnp.float32) + partial_ref[rows(c_l), hi]
            if s < N_DEV - 2:
                srs_r[s + 1, :, :] = acc_r.astype(jnp.bfloat16)
                srs_l[s + 1, :, :] = acc_l.astype(jnp.bfloat16)
            else:
                g_r = _gelu(acc_r)
                g_l = _gelu(acc_l)
                out_ref[rows(mod4(my + 1)), lo] = g_r
                out_ref[rows(mod4(my - 1)), hi] = g_l
                gsb_r[:, :] = g_r.astype(jnp.bfloat16)
                gsb_l[:, :] = g_l.astype(jnp.bfloat16)

        n_q = n_half // 2
        q0 = pl.ds(0, n_q)
        q1 = pl.ds(n_q, n_q)
        descs = []
        for h in range(N_DEV - 1):
            hop = []
            for (stage, ring_buf, sems_s, sems_r, dev) in (
                (gsb_r, ag_r, ag_r_send, ag_r_recv, right),
                (gsb_l, ag_l, ag_l_send, ag_l_recv, left),
            ):
                for qi, (qs, sem_base) in enumerate(((q0, 0), (q1, N_DEV - 1))):
                    hop.append(pltpu.make_async_remote_copy(
                        src_ref=(stage.at[:, qs] if h == 0
                                 else ring_buf.at[h - 1, :, qs]),
                        dst_ref=ring_buf.at[h, :, qs],
                        send_sem=sems_s.at[sem_base + h],
                        recv_sem=sems_r.at[sem_base + h],
                        device_id=(dev,), device_id_type=pl.DeviceIdType.MESH,
                    ))
            descs.append(hop)
        for d in descs[0]:
            d.start()
        for h in range(N_DEV - 1):
            for qi, d in enumerate(descs[h]):
                d.wait_recv()
                if h < N_DEV - 2:
                    descs[h + 1][qi].start()
            out_ref[rows(mod4(my - h)), lo] = ag_r[h, :, :].astype(jnp.float32)
            out_ref[rows(mod4(my + h)), hi] = ag_l[h, :, :].astype(jnp.float32)
        for hop in descs:
            for d in hop:
                d.wait_send()

    comm_shape = (N_DEV - 1, m_chunk, n_half)
    return pl.pallas_call(
        body,
        out_shape=jax.ShapeDtypeStruct((m, n), jnp.float32),
        in_specs=[
            pl.BlockSpec(memory_space=pltpu.VMEM),
            pl.BlockSpec(memory_space=pltpu.VMEM),
        ],
        out_specs=pl.BlockSpec(memory_space=pltpu.VMEM),
        scratch_shapes=[
            pltpu.VMEM((m, n), jnp.float32),
            pltpu.VMEM(comm_shape, jnp.bfloat16),
            pltpu.VMEM(comm_shape, jnp.bfloat16),
            pltpu.VMEM(comm_shape, jnp.bfloat16),
            pltpu.VMEM(comm_shape, jnp.bfloat16),
            pltpu.VMEM((m_chunk, n_half), jnp.bfloat16),
            pltpu.VMEM((m_chunk, n_half), jnp.bfloat16),
            pltpu.VMEM(comm_shape, jnp.bfloat16),
            pltpu.VMEM(comm_shape, jnp.bfloat16),
            pltpu.SemaphoreType.DMA((N_DEV - 1,)),
            pltpu.SemaphoreType.DMA((N_DEV - 1,)),
            pltpu.SemaphoreType.DMA((N_DEV - 1,)),
            pltpu.SemaphoreType.DMA((N_DEV - 1,)),
            pltpu.SemaphoreType.DMA((2 * (N_DEV - 1),)),
            pltpu.SemaphoreType.DMA((2 * (N_DEV - 1),)),
            pltpu.SemaphoreType.DMA((2 * (N_DEV - 1),)),
            pltpu.SemaphoreType.DMA((2 * (N_DEV - 1),)),
        ],
        compiler_params=pltpu.CompilerParams(collective_id=0),
    )(A, B)
